# Initial kernel scaffold; baseline (speedup 1.0000x reference)
import jax
import jax.numpy as jnp
from jax.experimental import pallas as pl


def kernel(fea, spa_mask, ch_mask, W0, W1, W2, W3, Wc, bc):
    raise NotImplementedError("write your pallas kernel here")


# speedup vs baseline: 1.6621x; 1.6621x over previous
import jax
import jax.numpy as jnp
from jax import lax
from jax.experimental import pallas as pl
from jax.experimental.pallas import tpu as pltpu

_TAU = 1.0
_NL = 4
_H = 224
_W = 224
_C = 96
_BH = 32
_NBLK = _H // _BH


def _smb_body(fea_ref, spa_ref, ch8_ref, g8_ref, chT_ref, gT_ref,
              w0_ref, w1_ref, w2_ref, w3_ref, wc_ref, bc_ref,
              y_ref, cm_ref, s0_ref, s1_ref):
    pid = pl.program_id(0)
    g0 = pid * _BH
    f32 = jnp.float32
    bf16 = jnp.bfloat16

    inv_tau = 1.0 / _TAU
    l8 = (ch8_ref[:, :] + g8_ref[:, :]) * inv_tau
    lT = (chT_ref[:, :] + gT_ref[:, :]) * inv_tau

    cmd_r, cms_r, cmdT, cmsT = [], [], [], []
    for i in range(_NL):
        a = l8[2 * i:2 * i + 1, :]
        b = l8[2 * i + 1:2 * i + 2, :]
        m = jnp.maximum(a, b)
        ea = jnp.exp(a - m)
        eb = jnp.exp(b - m)
        s = ea + eb
        cmd_r.append(ea / s)
        cms_r.append(eb / s)
        aT = lT[:, 2 * i:2 * i + 1]
        bT = lT[:, 2 * i + 1:2 * i + 2]
        mT = jnp.maximum(aT, bT)
        eaT = jnp.exp(aT - mT)
        ebT = jnp.exp(bT - mT)
        sT = eaT + ebT
        cmdT.append(eaT / sT)
        cmsT.append(ebT / sT)

    rows = []
    for i in range(_NL):
        rows.append(cmd_r[i])
        rows.append(cms_r[i])
    cm_ref[0, :, :] = jnp.concatenate(rows, axis=0)

    zcol = jnp.zeros((_BH + 6, 1, _C), dtype=bf16)
    s0_ref[:, 0:1, :] = zcol
    s0_ref[:, _W + 1:_W + 2, :] = zcol
    s1_ref[:, 0:1, :] = zcol
    s1_ref[:, _W + 1:_W + 2, :] = zcol

    spaw = spa_ref[pl.ds(g0, _BH + 8), :]

    def mults(layer, nrows):
        spal = spaw[layer:layer + nrows][:, :, None]
        ridx = lax.broadcasted_iota(jnp.int32, (nrows, _W), 0) + (g0 - 4 + layer)
        valid = ((ridx >= 0) & (ridx < _H)).astype(f32)[:, :, None]
        cs = cms_r[layer - 1].reshape(1, 1, _C)
        cd = cmd_r[layer - 1].reshape(1, 1, _C)
        return spal, spal * cs + valid * cd

    n1 = _BH + 6
    acc = jnp.zeros((n1 * _W, _C), dtype=f32)
    for dh in range(3):
        for dw in range(3):
            xs = fea_ref[pl.ds(g0 + dh, n1), pl.ds(dw, _W), :]
            acc = acc + jnp.dot(xs.reshape(n1 * _W, _C), w0_ref[dh, dw, :, :],
                                preferred_element_type=f32)
    spal, A = mults(1, n1)
    x = jnp.maximum(acc.reshape(n1, _W, _C) * A, 0.0)
    s0_ref[0:n1, 1:_W + 1, :] = x.astype(bf16)
    yacc = jnp.dot(x[3:3 + _BH].reshape(_BH * _W, _C).astype(bf16),
                   wc_ref[0, :, :], preferred_element_type=f32)

    wrefs = {2: w1_ref, 3: w2_ref, 4: w3_ref}
    for layer in range(2, _NL + 1):
        nout = _BH + 8 - 2 * layer
        sprev = s0_ref if layer % 2 == 0 else s1_ref
        scur = s1_ref if layer % 2 == 0 else s0_ref
        wref = wrefs[layer]
        cd_p = cmdT[layer - 2]
        cs_p = cmsT[layer - 2]
        acc = jnp.zeros((nout * _W, 2 * _C), dtype=f32)
        for dh in range(3):
            for dw in range(3):
                w = wref[dh, dw, :, :]
                wcat = jnp.concatenate([w * cd_p, w * cs_p], axis=1).astype(bf16)
                xs = sprev[dh:dh + nout, pl.ds(dw, _W), :]
                acc = acc + jnp.dot(xs.reshape(nout * _W, _C), wcat,
                                    preferred_element_type=f32)
        fd = acc[:, :_C].reshape(nout, _W, _C)
        fs = acc[:, _C:].reshape(nout, _W, _C)
        spal, A = mults(layer, nout)
        x = jnp.maximum(fd * A + fs * spal, 0.0)
        if layer < _NL:
            scur[0:nout, 1:_W + 1, :] = x.astype(bf16)
        yacc = yacc + jnp.dot(
            x[4 - layer:4 - layer + _BH].reshape(_BH * _W, _C).astype(bf16),
            wc_ref[layer - 1, :, :], preferred_element_type=f32)

    y_ref[:, :, :] = (yacc + bc_ref[:, :]).reshape(_BH, _W, _C)


def kernel(fea, spa_mask, ch_mask, W0, W1, W2, W3, Wc, bc):
    f32 = jnp.float32
    bf16 = jnp.bfloat16

    u = jax.random.uniform(jax.random.key(1234), ch_mask.shape,
                           minval=1e-8, maxval=1.0 - 1e-8, dtype=ch_mask.dtype)
    g = -jnp.log(-jnp.log(u))
    ch8 = ch_mask.reshape(2 * _NL, _C)
    g8 = g.reshape(2 * _NL, _C)
    chT = ch8.T
    gT = g8.T

    fea_p = jnp.pad(fea[0], ((4, 4), (1, 1), (0, 0))).astype(bf16)
    spa_p = jnp.pad(spa_mask[0, :, :, 0], ((4, 4), (0, 0)))
    w0 = W0.astype(bf16)
    wc = Wc.reshape(_NL, _C, _C).astype(bf16)
    bcr = bc.reshape(1, _C)

    y, cm8 = pl.pallas_call(
        _smb_body,
        grid=(_NBLK,),
        in_specs=[
            pl.BlockSpec(fea_p.shape, lambda i: (0, 0, 0)),
            pl.BlockSpec(spa_p.shape, lambda i: (0, 0)),
            pl.BlockSpec((2 * _NL, _C), lambda i: (0, 0)),
            pl.BlockSpec((2 * _NL, _C), lambda i: (0, 0)),
            pl.BlockSpec((_C, 2 * _NL), lambda i: (0, 0)),
            pl.BlockSpec((_C, 2 * _NL), lambda i: (0, 0)),
            pl.BlockSpec((3, 3, _C, _C), lambda i: (0, 0, 0, 0)),
            pl.BlockSpec((3, 3, _C, _C), lambda i: (0, 0, 0, 0)),
            pl.BlockSpec((3, 3, _C, _C), lambda i: (0, 0, 0, 0)),
            pl.BlockSpec((3, 3, _C, _C), lambda i: (0, 0, 0, 0)),
            pl.BlockSpec((_NL, _C, _C), lambda i: (0, 0, 0)),
            pl.BlockSpec((1, _C), lambda i: (0, 0)),
        ],
        out_specs=[
            pl.BlockSpec((_BH, _W, _C), lambda i: (i, 0, 0)),
            pl.BlockSpec((1, 2 * _NL, _C), lambda i: (i, 0, 0)),
        ],
        out_shape=[
            jax.ShapeDtypeStruct((_H, _W, _C), f32),
            jax.ShapeDtypeStruct((_NBLK, 2 * _NL, _C), f32),
        ],
        scratch_shapes=[
            pltpu.VMEM((_BH + 6, _W + 2, _C), bf16),
            pltpu.VMEM((_BH + 6, _W + 2, _C), bf16),
        ],
        compiler_params=pltpu.CompilerParams(
            dimension_semantics=("arbitrary",)),
    )(fea_p, spa_p, ch8, g8, chT, gT, w0, W1, W2, W3, wc, bcr)

    return y.reshape(1, _H, _W, _C), cm8[0].reshape(1, _NL, 2, _C)
